# baseline (speedup 1.0000x reference)
import functools

import jax
import jax.numpy as jnp
from jax import lax
from jax.experimental import pallas as pl
from jax.experimental.pallas import tpu as pltpu
from jax.experimental.pallas import tpu_sc as plsc

NC = 2
NS = 16
NW = NC * NS
NBUF = 2


@functools.lru_cache(maxsize=None)
def _build(batch, hist, vocab, dim):
    assert batch % NW == 0 and hist % NBUF == 0
    bw = batch // NW
    n_groups = hist // NBUF
    mesh = plsc.VectorSubcoreMesh(core_axis_name="c", subcore_axis_name="s")

    @functools.partial(
        pl.kernel,
        mesh=mesh,
        out_type=jax.ShapeDtypeStruct((hist, dim, batch), jnp.float32),
        scratch_types=(
            [pltpu.VMEM((bw,), jnp.int32) for _ in range(NBUF)]
            + [pltpu.VMEM((bw, dim), jnp.float32) for _ in range(NBUF)]
            + [pltpu.VMEM((dim, bw), jnp.float32) for _ in range(NBUF)]
            + [pltpu.SemaphoreType.DMA((NBUF,)),
               pltpu.SemaphoreType.DMA((NBUF,)),
               pltpu.SemaphoreType.DMA((NBUF,))]
        ),
        compiler_params=pltpu.CompilerParams(use_tc_tiling_on_sc=False, needs_layout_passes=False),
    )
    def gather_kernel(idx_hbm, table_hbm, out_hbm, *rest):
        idx_v = rest[:NBUF]
        gat_v = rest[NBUF:2 * NBUF]
        stg_v = rest[2 * NBUF:3 * NBUF]
        sem_i, sem_g, sem_s = rest[3 * NBUF:]
        wid = lax.axis_index("s") * NC + lax.axis_index("c")
        b0 = wid * bw
        iota16 = jax.lax.iota(jnp.int32, 16)

        for b in range(NBUF):
            pltpu.async_copy(
                idx_hbm.at[b, pl.ds(b0, bw)], idx_v[b], sem_i.at[b]
            )

        @pl.loop(0, n_groups)
        def _(g):
            s0 = g * NBUF
            for b in range(NBUF):
                @pl.when(g > 0)
                def _():
                    pltpu.make_async_copy(
                        stg_v[b], out_hbm.at[0].at[:, pl.ds(0, bw)],
                        sem_s.at[b],
                    ).wait()

                pltpu.make_async_copy(
                    idx_hbm.at[0, pl.ds(0, bw)], idx_v[b], sem_i.at[b]
                ).wait()
                pltpu.async_copy(
                    table_hbm.at[idx_v[b]], gat_v[b], sem_g.at[b]
                )
            for b in range(NBUF):
                s = s0 + b
                pltpu.make_async_copy(
                    table_hbm.at[idx_v[b]], gat_v[b], sem_g.at[b]
                ).wait()
                @pl.when(s + NBUF < hist)
                def _():
                    pltpu.async_copy(
                        idx_hbm.at[s + NBUF, pl.ds(b0, bw)], idx_v[b],
                        sem_i.at[b],
                    )

                @pl.loop(0, dim)
                def _(dd):
                    col = jnp.full((16,), dd, jnp.int32)
                    for r16 in range(bw // 16):
                        rows = iota16 + (r16 * 16)
                        vec = plsc.load_gather(gat_v[b], [rows, col])
                        stg_v[b][dd, pl.ds(r16 * 16, 16)] = vec

                pltpu.async_copy(
                    stg_v[b],
                    out_hbm.at[s].at[:, pl.ds(b0, bw)],
                    sem_s.at[b],
                )

        for b in range(NBUF):
            pltpu.make_async_copy(
                stg_v[b], out_hbm.at[0].at[:, pl.ds(0, bw)], sem_s.at[b]
            ).wait()

    return gather_kernel


def kernel(input_ids, table):
    batch, hist = input_ids.shape
    vocab, dim = table.shape
    out_t = _build(batch, hist, vocab, dim)(
        input_ids.astype(jnp.int32).T, table
    )
    return jnp.transpose(out_t, (2, 0, 1))


# speedup vs baseline: 1.4535x; 1.4535x over previous
import functools

import jax
import jax.numpy as jnp
from jax import lax
from jax.experimental import pallas as pl
from jax.experimental.pallas import tpu as pltpu
from jax.experimental.pallas import tpu_sc as plsc

NC = 2
NS = 16
NW = NC * NS
NBUF = 2
BPAD = 1


@functools.lru_cache(maxsize=None)
def _build_shuffle(batch, hist):
    bw = batch // NW
    mesh = plsc.VectorSubcoreMesh(core_axis_name="c", subcore_axis_name="s")

    @functools.partial(
        pl.kernel,
        mesh=mesh,
        out_type=jax.ShapeDtypeStruct((hist * batch,), jnp.int32),
        scratch_types=[
            pltpu.VMEM((hist, bw), jnp.int32),
            pltpu.SemaphoreType.DMA,
            pltpu.SemaphoreType.DMA,
        ],
        compiler_params=pltpu.CompilerParams(use_tc_tiling_on_sc=True,
                                             needs_layout_passes=False),
    )
    def shuffle_kernel(idx_hbm, out_hbm, blk, sem_i, sem_o):
        wid = lax.axis_index("s") * NC + lax.axis_index("c")
        b0 = wid * bw
        pltpu.async_copy(idx_hbm.at[:, pl.ds(b0, bw)], blk, sem_i).wait()
        for h in range(hist):
            pltpu.async_copy(
                blk.at[h], out_hbm.at[pl.ds(h * batch + b0, bw)], sem_o
            )
        for h in range(hist):
            pltpu.make_async_copy(
                blk.at[0], out_hbm.at[pl.ds(0, bw)], sem_o
            ).wait()

    return shuffle_kernel


@functools.lru_cache(maxsize=None)
def _build_gather(batch, hist, vocab, dim):
    assert batch % NW == 0 and hist % NBUF == 0
    bw = batch // NW
    bwp = bw + BPAD
    n_groups = hist // NBUF
    mesh = plsc.VectorSubcoreMesh(core_axis_name="c", subcore_axis_name="s")

    @functools.partial(
        pl.kernel,
        mesh=mesh,
        out_type=jax.ShapeDtypeStruct((hist, dim, batch), jnp.float32),
        scratch_types=(
            [pltpu.VMEM((bw,), jnp.int32) for _ in range(NBUF)]
            + [pltpu.VMEM((bw, dim), jnp.float32) for _ in range(NBUF)]
            + [pltpu.VMEM((dim, bwp), jnp.float32) for _ in range(NBUF)]
            + [pltpu.SemaphoreType.DMA((NBUF,)),
               pltpu.SemaphoreType.DMA((NBUF,)),
               pltpu.SemaphoreType.DMA((NBUF,))]
        ),
        compiler_params=pltpu.CompilerParams(use_tc_tiling_on_sc=False,
                                             needs_layout_passes=False),
    )
    def gather_kernel(idx_hbm, table_hbm, out_hbm, *rest):
        idx_v = rest[:NBUF]
        gat_v = rest[NBUF:2 * NBUF]
        stg_v = rest[2 * NBUF:3 * NBUF]
        sem_i, sem_g, sem_s = rest[3 * NBUF:]
        wid = lax.axis_index("s") * NC + lax.axis_index("c")
        b0 = wid * bw
        iota16 = jax.lax.iota(jnp.int32, 16)

        for b in range(NBUF):
            pltpu.async_copy(
                idx_hbm.at[pl.ds(b * batch + b0, bw)], idx_v[b], sem_i.at[b]
            )

        @pl.loop(0, n_groups)
        def _(g):
            s0 = g * NBUF
            for b in range(NBUF):
                @pl.when(g > 0)
                def _():
                    pltpu.make_async_copy(
                        stg_v[b].at[:, pl.ds(0, bw)],
                        out_hbm.at[0].at[:, pl.ds(0, bw)],
                        sem_s.at[b],
                    ).wait()

                pltpu.make_async_copy(
                    idx_hbm.at[pl.ds(0, bw)], idx_v[b], sem_i.at[b]
                ).wait()
                pltpu.async_copy(
                    table_hbm.at[idx_v[b]], gat_v[b], sem_g.at[b]
                )
            for b in range(NBUF):
                s = s0 + b
                pltpu.make_async_copy(
                    table_hbm.at[idx_v[b]], gat_v[b], sem_g.at[b]
                ).wait()
                @pl.when(s + NBUF < hist)
                def _():
                    pltpu.async_copy(
                        idx_hbm.at[pl.ds((s + NBUF) * batch + b0, bw)],
                        idx_v[b],
                        sem_i.at[b],
                    )

                @pl.loop(0, bw)
                def _(r):
                    rvec = jnp.full((16,), r, jnp.int32)
                    for c16 in range(dim // 16):
                        vec = gat_v[b][r, pl.ds(c16 * 16, 16)]
                        plsc.store_scatter(
                            stg_v[b], [iota16 + c16 * 16, rvec], vec
                        )

                pltpu.async_copy(
                    stg_v[b].at[:, pl.ds(0, bw)],
                    out_hbm.at[s].at[:, pl.ds(b0, bw)],
                    sem_s.at[b],
                )

        for b in range(NBUF):
            pltpu.make_async_copy(
                stg_v[b].at[:, pl.ds(0, bw)],
                out_hbm.at[0].at[:, pl.ds(0, bw)],
                sem_s.at[b],
            ).wait()

    return gather_kernel


def kernel(input_ids, table):
    batch, hist = input_ids.shape
    vocab, dim = table.shape
    ids_flat = _build_shuffle(batch, hist)(input_ids.astype(jnp.int32).T)
    out_t = _build_gather(batch, hist, vocab, dim)(ids_flat, table)
    return jnp.transpose(out_t, (2, 0, 1))


# speedup vs baseline: 1.5034x; 1.0343x over previous
import functools

import jax
import jax.numpy as jnp
from jax import lax
from jax.experimental import pallas as pl
from jax.experimental.pallas import tpu as pltpu
from jax.experimental.pallas import tpu_sc as plsc

NC = 2
NS = 16
NW = NC * NS
NBUF = 2
BPAD = 1


@functools.lru_cache(maxsize=None)
def _build_shuffle(batch, hist):
    bw = batch // NW
    mesh = plsc.VectorSubcoreMesh(core_axis_name="c", subcore_axis_name="s")

    @functools.partial(
        pl.kernel,
        mesh=mesh,
        out_type=jax.ShapeDtypeStruct((hist * batch,), jnp.int32),
        scratch_types=[
            pltpu.VMEM((hist, bw), jnp.int32),
            pltpu.SemaphoreType.DMA,
            pltpu.SemaphoreType.DMA,
        ],
        compiler_params=pltpu.CompilerParams(use_tc_tiling_on_sc=True,
                                             needs_layout_passes=False),
    )
    def shuffle_kernel(idx_hbm, out_hbm, blk, sem_i, sem_o):
        wid = lax.axis_index("s") * NC + lax.axis_index("c")
        b0 = wid * bw
        pltpu.async_copy(idx_hbm.at[:, pl.ds(b0, bw)], blk, sem_i).wait()
        for h in range(hist):
            pltpu.async_copy(
                blk.at[h], out_hbm.at[pl.ds(h * batch + b0, bw)], sem_o
            )
        for h in range(hist):
            pltpu.make_async_copy(
                blk.at[0], out_hbm.at[pl.ds(0, bw)], sem_o
            ).wait()

    return shuffle_kernel


@functools.lru_cache(maxsize=None)
def _build_gather(batch, hist, vocab, dim):
    assert batch % NW == 0 and hist % NBUF == 0
    bw = batch // NW
    bwp = bw + BPAD
    n_groups = hist // NBUF
    mesh = plsc.VectorSubcoreMesh(core_axis_name="c", subcore_axis_name="s")

    @functools.partial(
        pl.kernel,
        mesh=mesh,
        out_type=jax.ShapeDtypeStruct((hist, dim, batch), jnp.float32),
        scratch_types=(
            [pltpu.VMEM((bw,), jnp.int32) for _ in range(NBUF)]
            + [pltpu.VMEM((bw, dim), jnp.float32) for _ in range(NBUF)]
            + [pltpu.VMEM((dim, bwp), jnp.float32) for _ in range(NBUF)]
            + [pltpu.SemaphoreType.DMA((NBUF,)),
               pltpu.SemaphoreType.DMA((NBUF,)),
               pltpu.SemaphoreType.DMA((NBUF,))]
        ),
        compiler_params=pltpu.CompilerParams(use_tc_tiling_on_sc=False,
                                             needs_layout_passes=False),
    )
    def gather_kernel(idx_hbm, table_hbm, out_hbm, *rest):
        idx_v = rest[:NBUF]
        gat_v = rest[NBUF:2 * NBUF]
        stg_v = rest[2 * NBUF:3 * NBUF]
        sem_i, sem_g, sem_s = rest[3 * NBUF:]
        wid = lax.axis_index("s") * NC + lax.axis_index("c")
        b0 = wid * bw
        iota16 = jax.lax.iota(jnp.int32, 16)

        for b in range(NBUF):
            pltpu.async_copy(
                idx_hbm.at[pl.ds(b * batch + b0, bw)], idx_v[b], sem_i.at[b]
            )

        @pl.loop(0, n_groups)
        def _(g):
            s0 = g * NBUF
            for b in range(NBUF):
                @pl.when(g > 0)
                def _():
                    pltpu.make_async_copy(
                        stg_v[b].at[:, pl.ds(0, bw)],
                        out_hbm.at[0].at[:, pl.ds(0, bw)],
                        sem_s.at[b],
                    ).wait()

                pltpu.make_async_copy(
                    idx_hbm.at[pl.ds(0, bw)], idx_v[b], sem_i.at[b]
                ).wait()
                pltpu.async_copy(
                    table_hbm.at[idx_v[b]], gat_v[b], sem_g.at[b]
                )
            for b in range(NBUF):
                s = s0 + b
                pltpu.make_async_copy(
                    table_hbm.at[idx_v[b]], gat_v[b], sem_g.at[b]
                ).wait()
                @pl.when(s + NBUF < hist)
                def _():
                    pltpu.async_copy(
                        idx_hbm.at[pl.ds((s + NBUF) * batch + b0, bw)],
                        idx_v[b],
                        sem_i.at[b],
                    )

                @pl.loop(0, bw, unroll=8)
                def _(r):
                    rvec = jnp.full((16,), r, jnp.int32)
                    for c16 in range(dim // 16):
                        vec = gat_v[b][r, pl.ds(c16 * 16, 16)]
                        plsc.store_scatter(
                            stg_v[b], [iota16 + c16 * 16, rvec], vec
                        )

                pltpu.async_copy(
                    stg_v[b].at[:, pl.ds(0, bw)],
                    out_hbm.at[s].at[:, pl.ds(b0, bw)],
                    sem_s.at[b],
                )

        for b in range(NBUF):
            pltpu.make_async_copy(
                stg_v[b].at[:, pl.ds(0, bw)],
                out_hbm.at[0].at[:, pl.ds(0, bw)],
                sem_s.at[b],
            ).wait()

    return gather_kernel


def kernel(input_ids, table):
    batch, hist = input_ids.shape
    vocab, dim = table.shape
    ids_flat = _build_shuffle(batch, hist)(input_ids.astype(jnp.int32).T)
    out_t = _build_gather(batch, hist, vocab, dim)(ids_flat, table)
    return jnp.transpose(out_t, (2, 0, 1))
